# baseline (speedup 1.0000x reference)
import functools

import jax
import jax.numpy as jnp
from jax import lax
from jax.experimental import pallas as pl
from jax.experimental.pallas import tpu as pltpu
from jax.experimental.pallas import tpu_sc as plsc

B = 256
N = 65536
K = 10
BN = 8192
G = N // BN

NC = 2
NS = 16
L = 16
NW = NC * NS
CHUNK = N // NW
GROUPS = CHUNK // L


def _sc_prep_body(rates_hbm, assoc_hbm, occp_hbm, rbuf, abuf, obuf):
    wid = lax.axis_index("s") * NC + lax.axis_index("c")
    base = wid * CHUNK
    pltpu.sync_copy(rates_hbm.at[pl.ds(base, CHUNK), :], rbuf)

    lane = lax.iota(jnp.int32, L)
    zero = jnp.zeros((L,), jnp.float32)

    @plsc.parallel_loop(0, GROUPS, unroll=4, carry=(zero,) * K)
    def group(i, counts):
        rowbase = i * L
        rows = rowbase + lane
        vals = [
            plsc.load_gather(rbuf, [rows, jnp.full((L,), k, jnp.int32)])
            for k in range(K)
        ]
        denom = jnp.abs(vals[0])
        for k in range(1, K):
            denom = denom + jnp.abs(vals[k])
        denom = jnp.maximum(denom, 1e-12)
        m = vals[0]
        for k in range(1, K):
            m = jnp.maximum(m, vals[k])
        w = m / denom
        amax = jnp.full((L,), K, jnp.int32)
        for k in range(K - 1, -1, -1):
            amax = jnp.where(vals[k] == m, k, amax)
        new_counts = []
        for k in range(K):
            hit = amax == k
            abuf[k, pl.ds(rowbase, L)] = jnp.where(hit, w, 0.0)
            new_counts.append(counts[k] + jnp.where(hit, 1.0, 0.0))
        return tuple(new_counts)

    counts = group

    occ = jnp.zeros((L,), jnp.float32)
    for k in range(K):
        occ = jnp.where(lane == k, jnp.sum(counts[k]), occ)
    obuf[...] = occ

    pltpu.sync_copy(abuf, assoc_hbm.at[:, pl.ds(base, CHUNK)])
    pltpu.sync_copy(obuf, occp_hbm.at[wid])


_sc_prep = pl.kernel(
    _sc_prep_body,
    out_type=(
        jax.ShapeDtypeStruct((K, N), jnp.float32),
        jax.ShapeDtypeStruct((NW, L), jnp.float32),
    ),
    mesh=plsc.VectorSubcoreMesh(core_axis_name="c", subcore_axis_name="s"),
    compiler_params=pltpu.CompilerParams(
        needs_layout_passes=False, use_tc_tiling_on_sc=False),
    scratch_types=[
        pltpu.VMEM((CHUNK, K), jnp.float32),
        pltpu.VMEM((K, CHUNK), jnp.float32),
        pltpu.VMEM((L,), jnp.float32),
    ],
)


def _tc_body(x_ref, a_ref, occp_ref, o_ref):
    i = pl.program_id(0)

    @pl.when(i == 0)
    def _init():
        o_ref[...] = jnp.zeros_like(o_ref)

    x = x_ref[...].astype(jnp.bfloat16)
    assoc = a_ref[...].astype(jnp.bfloat16)
    o_ref[...] += jax.lax.dot_general(
        x, assoc,
        dimension_numbers=(((1,), (1,)), ((), ())),
        preferred_element_type=jnp.float32,
    )

    @pl.when(i == G - 1)
    def _finish():
        occ = jnp.sum(occp_ref[...], axis=0, keepdims=True)[:, :K]
        y = o_ref[...]
        o_ref[...] = jnp.where(occ > 0.0, y / occ, 0.0)


@jax.jit
def kernel(inputs, rates):
    assoc, occp = _sc_prep(rates)
    out = pl.pallas_call(
        _tc_body,
        grid=(G,),
        in_specs=[
            pl.BlockSpec((B, BN), lambda i: (0, i)),
            pl.BlockSpec((K, BN), lambda i: (0, i)),
            pl.BlockSpec((NW, L), lambda i: (0, 0)),
        ],
        out_specs=pl.BlockSpec((B, K), lambda i: (0, 0)),
        out_shape=jax.ShapeDtypeStruct((B, K), jnp.float32),
        compiler_params=pltpu.CompilerParams(
            dimension_semantics=("arbitrary",),
        ),
    )(inputs, assoc, occp)
    return out


# speedup vs baseline: 5.3061x; 5.3061x over previous
import functools

import jax
import jax.numpy as jnp
from jax.experimental import pallas as pl
from jax.experimental.pallas import tpu as pltpu

B = 256
N = 65536
K = 10
BN = 8192
G = N // BN


def _body(x_ref, rt_ref, o_ref, occ_ref):
    i = pl.program_id(0)

    @pl.when(i == 0)
    def _init():
        o_ref[...] = jnp.zeros_like(o_ref)
        occ_ref[...] = jnp.zeros_like(occ_ref)

    r = rt_ref[...]
    denom = jnp.maximum(jnp.sum(jnp.abs(r), axis=0, keepdims=True), 1e-12)
    p = r / denom
    m = jnp.max(p, axis=0, keepdims=True)
    row = jax.lax.broadcasted_iota(jnp.int32, p.shape, 0)
    ismax = p == m
    amax = jnp.min(jnp.where(ismax, row, K), axis=0, keepdims=True)
    onehot = row == amax
    assoc = jnp.where(onehot, p, 0.0)

    x = x_ref[...].astype(jnp.bfloat16)
    part = jax.lax.dot_general(
        x, assoc.astype(jnp.bfloat16),
        dimension_numbers=(((1,), (1,)), ((), ())),
        preferred_element_type=jnp.float32,
    )
    o_ref[...] += part
    occ_ref[0:1, :] += jnp.sum(onehot.astype(jnp.float32), axis=1)[None, :]

    @pl.when(i == G - 1)
    def _finish():
        occ = occ_ref[0:1, :]
        y = o_ref[...]
        o_ref[...] = jnp.where(occ > 0.0, y / occ, 0.0)


@jax.jit
def kernel(inputs, rates):
    rates_t = rates.T
    out = pl.pallas_call(
        _body,
        grid=(G,),
        in_specs=[
            pl.BlockSpec((B, BN), lambda i: (0, i)),
            pl.BlockSpec((K, BN), lambda i: (0, i)),
        ],
        out_specs=pl.BlockSpec((B, K), lambda i: (0, 0)),
        out_shape=jax.ShapeDtypeStruct((B, K), jnp.float32),
        scratch_shapes=[pltpu.VMEM((1, K), jnp.float32)],
        compiler_params=pltpu.CompilerParams(
            dimension_semantics=("arbitrary",),
            allow_input_fusion=[False, True],
        ),
    )(inputs, rates_t)
    return out


# speedup vs baseline: 5.3289x; 1.0043x over previous
import jax
import jax.numpy as jnp
from jax.experimental import pallas as pl
from jax.experimental.pallas import tpu as pltpu

B = 256
N = 65536
K = 10
BN = 8192
G = N // BN


def _body(x_ref, rt_ref, o_ref, occ_ref):
    i = pl.program_id(0)

    @pl.when(i == 0)
    def _init():
        o_ref[...] = jnp.zeros_like(o_ref)
        occ_ref[...] = jnp.zeros_like(occ_ref)

    r = rt_ref[...]
    denom = jnp.maximum(jnp.sum(jnp.abs(r), axis=0, keepdims=True), 1e-12)
    p = r / denom
    m = jnp.max(p, axis=0, keepdims=True)
    row = jax.lax.broadcasted_iota(jnp.int32, p.shape, 0)
    ismax = p == m
    amax = jnp.min(jnp.where(ismax, row, K), axis=0, keepdims=True)
    onehot = row == amax
    assoc = jnp.where(onehot, p, 0.0)

    x = x_ref[...].astype(jnp.bfloat16)
    part = jax.lax.dot_general(
        x, assoc.astype(jnp.bfloat16),
        dimension_numbers=(((1,), (1,)), ((), ())),
        preferred_element_type=jnp.float32,
    )
    o_ref[...] += part
    occ_ref[0:1, :] += jnp.sum(onehot.astype(jnp.float32), axis=1)[None, :]

    @pl.when(i == G - 1)
    def _finish():
        occ = occ_ref[0:1, :]
        y = o_ref[...]
        o_ref[...] = jnp.where(occ > 0.0, y / occ, 0.0)


@jax.jit
def kernel(inputs, rates):
    rates_t = rates.T
    out = pl.pallas_call(
        _body,
        grid=(G,),
        in_specs=[
            pl.BlockSpec((B, BN), lambda i: (0, i)),
            pl.BlockSpec((K, BN), lambda i: (0, i)),
        ],
        out_specs=pl.BlockSpec((B, K), lambda i: (0, 0)),
        out_shape=jax.ShapeDtypeStruct((B, K), jnp.float32),
        scratch_shapes=[pltpu.VMEM((1, K), jnp.float32)],
        compiler_params=pltpu.CompilerParams(
            dimension_semantics=("arbitrary",),
            allow_input_fusion=[False, True],
        ),
    )(inputs, rates_t)
    return out
